# Initial kernel scaffold; baseline (speedup 1.0000x reference)
import jax
import jax.numpy as jnp
from jax.experimental import pallas as pl


def kernel(input, W0, W1, W2):
    raise NotImplementedError("write your pallas kernel here")


# speedup vs baseline: 1.0582x; 1.0582x over previous
import functools

import jax
import jax.numpy as jnp
from jax import lax
from jax.experimental import pallas as pl
from jax.experimental.pallas import tpu as pltpu
from jax.experimental.pallas import tpu_sc as plsc

_N = 32768
_D = 256
_T = 512
_CH = 128
_NW = 32
_BPW = _N // _NW


def _tc_stage_body(xn_ref, r_ref, w_ref, idx_ref, msum_ref):
    i = pl.program_id(0)

    @pl.when(i == 0)
    def _():
        msum_ref[0] = 0.0

    r = r_ref[...]
    W = w_ref[...]
    wn = jnp.sum(W * W, axis=1)
    s = jax.lax.dot_general(r, W, (((1,), (1,)), ((), ())),
                            preferred_element_type=jnp.float32)
    d2 = (xn_ref[...] - 2.0 * s) + wn[None, :]
    idx_ref[...] = jnp.argmin(d2, axis=1)[:, None].astype(jnp.int32)
    msum_ref[0] += jnp.sum(jnp.min(d2, axis=1))


def _tc_stage(r, xn, W):
    K = W.shape[0]
    idx, msum = pl.pallas_call(
        _tc_stage_body,
        grid=(_N // _T,),
        in_specs=[
            pl.BlockSpec((_T, 1), lambda i: (i, 0)),
            pl.BlockSpec((_T, _D), lambda i: (i, 0)),
            pl.BlockSpec((K, _D), lambda i: (0, 0)),
        ],
        out_specs=[
            pl.BlockSpec((_T, 1), lambda i: (i, 0)),
            pl.BlockSpec(memory_space=pltpu.SMEM),
        ],
        out_shape=[
            jax.ShapeDtypeStruct((_N, 1), jnp.int32),
            jax.ShapeDtypeStruct((1,), jnp.float32),
        ],
    )(xn, r, W)
    return idx.reshape(_N), msum[0]


def _sc_residual(r, idx, W):
    mesh = plsc.VectorSubcoreMesh(core_axis_name="c", subcore_axis_name="s")

    @functools.partial(
        pl.kernel, mesh=mesh,
        out_type=jax.ShapeDtypeStruct((_N, _D), jnp.float32),
        scratch_types=[
            pltpu.VMEM((_CH,), jnp.int32),
            pltpu.VMEM((_CH, _D), jnp.float32),
            pltpu.VMEM((_CH, _D), jnp.float32),
            pltpu.SemaphoreType.DMA,
        ],
    )
    def k(r_hbm, idx_hbm, w_hbm, out_hbm, idx_v, q_v, r_v, sem):
        wid = lax.axis_index("s") * 2 + lax.axis_index("c")
        base = wid * _BPW

        def chunk(c, carry):
            b = base + c * _CH
            pltpu.sync_copy(idx_hbm.at[pl.ds(b, _CH)], idx_v)
            pltpu.async_copy(w_hbm.at[idx_v], q_v, sem).wait()
            pltpu.sync_copy(r_hbm.at[pl.ds(b, _CH)], r_v)

            def row(t, carry2):
                for l in range(_D // 16):
                    sl = pl.ds(l * 16, 16)
                    r_v[t, sl] = r_v[t, sl] - q_v[t, sl]
                return carry2

            lax.fori_loop(0, _CH, row, 0)
            pltpu.sync_copy(r_v, out_hbm.at[pl.ds(b, _CH)])
            return carry

        lax.fori_loop(0, _BPW // _CH, chunk, 0)

    return k(r, idx, W)


def _sc_output(r2, x, idx, W):
    mesh = plsc.VectorSubcoreMesh(core_axis_name="c", subcore_axis_name="s")

    @functools.partial(
        pl.kernel, mesh=mesh,
        out_type=jax.ShapeDtypeStruct((_N, _D), jnp.float32),
        scratch_types=[
            pltpu.VMEM((_CH,), jnp.int32),
            pltpu.VMEM((_CH, _D), jnp.float32),
            pltpu.VMEM((_CH, _D), jnp.float32),
            pltpu.VMEM((_CH, _D), jnp.float32),
            pltpu.SemaphoreType.DMA,
        ],
    )
    def k(r_hbm, x_hbm, idx_hbm, w_hbm, out_hbm, idx_v, q_v, r_v, x_v, sem):
        wid = lax.axis_index("s") * 2 + lax.axis_index("c")
        base = wid * _BPW

        def chunk(c, carry):
            b = base + c * _CH
            pltpu.sync_copy(idx_hbm.at[pl.ds(b, _CH)], idx_v)
            pltpu.async_copy(w_hbm.at[idx_v], q_v, sem).wait()
            pltpu.sync_copy(r_hbm.at[pl.ds(b, _CH)], r_v)
            pltpu.sync_copy(x_hbm.at[pl.ds(b, _CH)], x_v)

            def row(t, carry2):
                for l in range(_D // 16):
                    sl = pl.ds(l * 16, 16)
                    x_v[t, sl] = x_v[t, sl] - (r_v[t, sl] - q_v[t, sl])
                return carry2

            lax.fori_loop(0, _CH, row, 0)
            pltpu.sync_copy(x_v, out_hbm.at[pl.ds(b, _CH)])
            return carry

        lax.fori_loop(0, _BPW // _CH, chunk, 0)

    return k(r2, x, idx, W)


def kernel(input, W0, W1, W2):
    xn0 = jnp.sum(input * input, axis=1, keepdims=True)
    idx0, m0 = _tc_stage(input, xn0, W0)
    r1 = _sc_residual(input, idx0, W0)

    xn1 = jnp.sum(r1 * r1, axis=1, keepdims=True)
    idx1, m1 = _tc_stage(r1, xn1, W1)
    r2 = _sc_residual(r1, idx1, W1)

    xn2 = jnp.sum(r2 * r2, axis=1, keepdims=True)
    idx2, m2 = _tc_stage(r2, xn2, W2)
    out = _sc_output(r2, input, idx2, W2)

    a = jnp.stack([m0, m1, m2]) * (1.0 / (_N * _D))
    losses = 1.0 * a + 0.25 * a
    return (out, losses)


# speedup vs baseline: 1.1166x; 1.0552x over previous
import functools

import jax
import jax.numpy as jnp
from jax import lax
from jax.experimental import pallas as pl
from jax.experimental.pallas import tpu as pltpu
from jax.experimental.pallas import tpu_sc as plsc

_N = 32768
_D = 256
_T = 512
_CH = 128
_NW = 32
_BPW = _N // _NW


def _tc_stage_body(xn_ref, wn_ref, r_ref, w_ref, idx_ref, msum_ref):
    i = pl.program_id(0)

    @pl.when(i == 0)
    def _():
        msum_ref[0] = 0.0

    r = r_ref[...]
    W = w_ref[...]
    s = jax.lax.dot_general(r, W, (((1,), (1,)), ((), ())),
                            preferred_element_type=jnp.float32)
    d2 = (xn_ref[...] - 2.0 * s) + wn_ref[...]
    K = W.shape[0]
    m = jnp.min(d2, axis=1, keepdims=True)
    iota = jax.lax.broadcasted_iota(jnp.int32, (_T, K), 1)
    idx = jnp.min(jnp.where(d2 == m, iota, K), axis=1)
    idx_ref[...] = idx[:, None]
    msum_ref[0] += jnp.sum(m)


def _tc_stage(r, xn, W):
    K = W.shape[0]
    wn = jnp.sum(W * W, axis=1)[None, :]
    idx, msum = pl.pallas_call(
        _tc_stage_body,
        grid=(_N // _T,),
        in_specs=[
            pl.BlockSpec((_T, 1), lambda i: (i, 0)),
            pl.BlockSpec((1, K), lambda i: (0, 0)),
            pl.BlockSpec((_T, _D), lambda i: (i, 0)),
            pl.BlockSpec((K, _D), lambda i: (0, 0)),
        ],
        out_specs=[
            pl.BlockSpec((_T, 1), lambda i: (i, 0)),
            pl.BlockSpec(memory_space=pltpu.SMEM),
        ],
        out_shape=[
            jax.ShapeDtypeStruct((_N, 1), jnp.int32),
            jax.ShapeDtypeStruct((1,), jnp.float32),
        ],
    )(xn, wn, r, W)
    return idx.reshape(_N), msum[0]


def _sc_residual(r, idx, W):
    mesh = plsc.VectorSubcoreMesh(core_axis_name="c", subcore_axis_name="s")

    @functools.partial(
        pl.kernel, mesh=mesh,
        out_type=jax.ShapeDtypeStruct((_N, _D), jnp.float32),
        scratch_types=[
            pltpu.VMEM((_CH,), jnp.int32),
            pltpu.VMEM((_CH, _D), jnp.float32),
            pltpu.VMEM((_CH, _D), jnp.float32),
            pltpu.SemaphoreType.DMA,
        ],
    )
    def k(r_hbm, idx_hbm, w_hbm, out_hbm, idx_v, q_v, r_v, sem):
        wid = lax.axis_index("s") * 2 + lax.axis_index("c")
        base = wid * _BPW

        def chunk(c, carry):
            b = base + c * _CH
            pltpu.sync_copy(idx_hbm.at[pl.ds(b, _CH)], idx_v)
            pltpu.async_copy(w_hbm.at[idx_v], q_v, sem).wait()
            pltpu.sync_copy(r_hbm.at[pl.ds(b, _CH)], r_v)

            def row(t, carry2):
                for l in range(_D // 16):
                    sl = pl.ds(l * 16, 16)
                    r_v[t, sl] = r_v[t, sl] - q_v[t, sl]
                return carry2

            lax.fori_loop(0, _CH, row, 0)
            pltpu.sync_copy(r_v, out_hbm.at[pl.ds(b, _CH)])
            return carry

        lax.fori_loop(0, _BPW // _CH, chunk, 0)

    return k(r, idx, W)


def _sc_output(r2, x, idx, W):
    mesh = plsc.VectorSubcoreMesh(core_axis_name="c", subcore_axis_name="s")

    @functools.partial(
        pl.kernel, mesh=mesh,
        out_type=jax.ShapeDtypeStruct((_N, _D), jnp.float32),
        scratch_types=[
            pltpu.VMEM((_CH,), jnp.int32),
            pltpu.VMEM((_CH, _D), jnp.float32),
            pltpu.VMEM((_CH, _D), jnp.float32),
            pltpu.VMEM((_CH, _D), jnp.float32),
            pltpu.SemaphoreType.DMA,
        ],
    )
    def k(r_hbm, x_hbm, idx_hbm, w_hbm, out_hbm, idx_v, q_v, r_v, x_v, sem):
        wid = lax.axis_index("s") * 2 + lax.axis_index("c")
        base = wid * _BPW

        def chunk(c, carry):
            b = base + c * _CH
            pltpu.sync_copy(idx_hbm.at[pl.ds(b, _CH)], idx_v)
            pltpu.async_copy(w_hbm.at[idx_v], q_v, sem).wait()
            pltpu.sync_copy(r_hbm.at[pl.ds(b, _CH)], r_v)
            pltpu.sync_copy(x_hbm.at[pl.ds(b, _CH)], x_v)

            def row(t, carry2):
                for l in range(_D // 16):
                    sl = pl.ds(l * 16, 16)
                    x_v[t, sl] = x_v[t, sl] - (r_v[t, sl] - q_v[t, sl])
                return carry2

            lax.fori_loop(0, _CH, row, 0)
            pltpu.sync_copy(x_v, out_hbm.at[pl.ds(b, _CH)])
            return carry

        lax.fori_loop(0, _BPW // _CH, chunk, 0)

    return k(r2, x, idx, W)


def kernel(input, W0, W1, W2):
    xn0 = jnp.sum(input * input, axis=1, keepdims=True)
    idx0, m0 = _tc_stage(input, xn0, W0)
    r1 = _sc_residual(input, idx0, W0)

    xn1 = jnp.sum(r1 * r1, axis=1, keepdims=True)
    idx1, m1 = _tc_stage(r1, xn1, W1)
    r2 = _sc_residual(r1, idx1, W1)

    xn2 = jnp.sum(r2 * r2, axis=1, keepdims=True)
    idx2, m2 = _tc_stage(r2, xn2, W2)
    out = _sc_output(r2, input, idx2, W2)

    a = jnp.stack([m0, m1, m2]) * (1.0 / (_N * _D))
    losses = 1.0 * a + 0.25 * a
    return (out, losses)


# speedup vs baseline: 1.5598x; 1.3969x over previous
import functools

import jax
import jax.numpy as jnp
from jax import lax
from jax.experimental import pallas as pl
from jax.experimental.pallas import tpu as pltpu
from jax.experimental.pallas import tpu_sc as plsc

_N = 32768
_D = 256
_T = 512
_NB = _N // _T
_CH = 128
_NW = 32
_BPW = _N // _NW


def _row_norm(r):
    xx = r * r
    h = xx[:, :128] + xx[:, 128:]
    ht = jnp.transpose(h)
    acc = ht[0:8]
    for j in range(1, 16):
        acc = acc + ht[8 * j:8 * j + 8]
    p = acc[0:4] + acc[4:8]
    q = p[0:2] + p[2:4]
    return q[0:1] + q[1:2]


def _make_tc_body(nq, K):
    def body(*refs):
        wn_ref, x_ref = refs[0], refs[1]
        q_refs = refs[2:2 + nq]
        w_ref = refs[2 + nq]
        idx_ref, msum_ref = refs[3 + nq], refs[4 + nq]
        i = pl.program_id(0)

        @pl.when(i == 0)
        def _():
            msum_ref[0] = 0.0

        r = x_ref[...]
        for q_ref in q_refs:
            r = r - q_ref[...]
        xn = _row_norm(r)
        st = jax.lax.dot_general(w_ref[...], r, (((1,), (1,)), ((), ())),
                                 preferred_element_type=jnp.float32)
        d2 = (xn - 2.0 * st) + wn_ref[...]
        m = jnp.min(d2, axis=0, keepdims=True)
        iota = jax.lax.broadcasted_iota(jnp.int32, (K, _T), 0)
        idx = jnp.min(jnp.where(d2 == m, iota, K), axis=0, keepdims=True)
        idx_ref[...] = idx[None]
        msum_ref[0] += jnp.sum(m)

    return body


def _tc_stage(x, qs, W):
    K = W.shape[0]
    wn = jnp.sum(W * W, axis=1)[:, None]
    nq = len(qs)
    idx, msum = pl.pallas_call(
        _make_tc_body(nq, K),
        grid=(_NB,),
        in_specs=[
            pl.BlockSpec((K, 1), lambda i: (0, 0)),
            pl.BlockSpec((_T, _D), lambda i: (i, 0)),
        ] + [
            pl.BlockSpec((_T, _D), lambda i: (i, 0)) for _ in range(nq)
        ] + [
            pl.BlockSpec((K, _D), lambda i: (0, 0)),
        ],
        out_specs=[
            pl.BlockSpec((1, 1, _T), lambda i: (i, 0, 0)),
            pl.BlockSpec(memory_space=pltpu.SMEM),
        ],
        out_shape=[
            jax.ShapeDtypeStruct((_NB, 1, _T), jnp.int32),
            jax.ShapeDtypeStruct((1,), jnp.float32),
        ],
    )(wn, x, *qs, W)
    return idx.reshape(_N), msum[0]


def _sc_gather(idx, W):
    mesh = plsc.VectorSubcoreMesh(core_axis_name="c", subcore_axis_name="s")

    @functools.partial(
        pl.kernel, mesh=mesh,
        out_type=jax.ShapeDtypeStruct((_N, _D), jnp.float32),
        scratch_types=[
            pltpu.VMEM((_CH,), jnp.int32),
            pltpu.VMEM((_CH, _D), jnp.float32),
            pltpu.SemaphoreType.DMA,
        ],
    )
    def k(idx_hbm, w_hbm, out_hbm, idx_v, q_v, sem):
        wid = lax.axis_index("s") * 2 + lax.axis_index("c")
        base = wid * _BPW

        def chunk(c, carry):
            b = base + c * _CH
            pltpu.sync_copy(idx_hbm.at[pl.ds(b, _CH)], idx_v)
            pltpu.async_copy(w_hbm.at[idx_v], q_v, sem).wait()
            pltpu.sync_copy(q_v, out_hbm.at[pl.ds(b, _CH)])
            return carry

        lax.fori_loop(0, _BPW // _CH, chunk, 0)

    return k(idx, W)


def kernel(input, W0, W1, W2):
    idx0, m0 = _tc_stage(input, [], W0)
    q0 = _sc_gather(idx0, W0)

    idx1, m1 = _tc_stage(input, [q0], W1)
    q1 = _sc_gather(idx1, W1)

    idx2, m2 = _tc_stage(input, [q0, q1], W2)
    q2 = _sc_gather(idx2, W2)

    out = (q0 + q1) + q2
    a = jnp.stack([m0, m1, m2]) * (1.0 / (_N * _D))
    losses = 1.0 * a + 0.25 * a
    return (out, losses)


# speedup vs baseline: 1.5653x; 1.0035x over previous
import functools

import jax
import jax.numpy as jnp
from jax import lax
from jax.experimental import pallas as pl
from jax.experimental.pallas import tpu as pltpu
from jax.experimental.pallas import tpu_sc as plsc

_N = 32768
_D = 256
_T = 512
_NB = _N // _T
_CH = 128
_NW = 32
_BPW = _N // _NW


def _row_norm(r):
    xx = r * r
    h = xx[:, :128] + xx[:, 128:]
    ht = jnp.transpose(h)
    acc = ht[0:8]
    for j in range(1, 16):
        acc = acc + ht[8 * j:8 * j + 8]
    p = acc[0:4] + acc[4:8]
    q = p[0:2] + p[2:4]
    return q[0:1] + q[1:2]


def _make_tc_body(nq, K):
    def body(*refs):
        wn_ref, x_ref = refs[0], refs[1]
        q_refs = refs[2:2 + nq]
        w_ref = refs[2 + nq]
        idx_ref, msum_ref = refs[3 + nq], refs[4 + nq]
        i = pl.program_id(0)

        @pl.when(i == 0)
        def _():
            msum_ref[0] = 0.0

        r = x_ref[...]
        for q_ref in q_refs:
            r = r - q_ref[...]
        xn = _row_norm(r)
        st = jax.lax.dot_general(w_ref[...], r, (((1,), (1,)), ((), ())),
                                 preferred_element_type=jnp.float32)
        d2 = (xn - 2.0 * st) + wn_ref[...]
        m = jnp.min(d2, axis=0, keepdims=True)
        iota = jax.lax.broadcasted_iota(jnp.int32, (K, _T), 0)
        idx = jnp.min(jnp.where(d2 == m, iota, K), axis=0, keepdims=True)
        idx_ref[...] = idx[None]
        msum_ref[0] += jnp.sum(m)

    return body


def _tc_stage(x, qs, W):
    ntok = x.shape[0]
    K = W.shape[0]
    wn = jnp.sum(W * W, axis=1)[:, None]
    nq = len(qs)
    nb = ntok // _T
    idx, msum = pl.pallas_call(
        _make_tc_body(nq, K),
        grid=(nb,),
        in_specs=[
            pl.BlockSpec((K, 1), lambda i: (0, 0)),
            pl.BlockSpec((_T, _D), lambda i: (i, 0)),
        ] + [
            pl.BlockSpec((_T, _D), lambda i: (i, 0)) for _ in range(nq)
        ] + [
            pl.BlockSpec((K, _D), lambda i: (0, 0)),
        ],
        out_specs=[
            pl.BlockSpec((1, 1, _T), lambda i: (i, 0, 0)),
            pl.BlockSpec(memory_space=pltpu.SMEM),
        ],
        out_shape=[
            jax.ShapeDtypeStruct((nb, 1, _T), jnp.int32),
            jax.ShapeDtypeStruct((1,), jnp.float32),
        ],
    )(wn, x, *qs, W)
    return idx.reshape(ntok), msum[0]


def _sc_gather(idx, W):
    ntok = idx.shape[0]
    bpw = ntok // _NW
    mesh = plsc.VectorSubcoreMesh(core_axis_name="c", subcore_axis_name="s")

    @functools.partial(
        pl.kernel, mesh=mesh,
        out_type=jax.ShapeDtypeStruct((ntok, _D), jnp.float32),
        scratch_types=[
            pltpu.VMEM((_CH,), jnp.int32),
            pltpu.VMEM((_CH, _D), jnp.float32),
            pltpu.SemaphoreType.DMA,
        ],
    )
    def k(idx_hbm, w_hbm, out_hbm, idx_v, q_v, sem):
        wid = lax.axis_index("s") * 2 + lax.axis_index("c")
        base = wid * bpw

        def chunk(c, carry):
            b = base + c * _CH
            pltpu.sync_copy(idx_hbm.at[pl.ds(b, _CH)], idx_v)
            pltpu.async_copy(w_hbm.at[idx_v], q_v, sem).wait()
            pltpu.sync_copy(q_v, out_hbm.at[pl.ds(b, _CH)])
            return carry

        lax.fori_loop(0, bpw // _CH, chunk, 0)

    return k(idx, W)


_H = 2


def kernel(input, W0, W1, W2):
    nh = _N // _H
    xs = [jax.lax.slice(input, (h * nh, 0), ((h + 1) * nh, _D))
          for h in range(_H)]
    outs, msums = [], [jnp.float32(0.0)] * 3
    for x in xs:
        idx0, m0 = _tc_stage(x, [], W0)
        q0 = _sc_gather(idx0, W0)
        idx1, m1 = _tc_stage(x, [q0], W1)
        q1 = _sc_gather(idx1, W1)
        idx2, m2 = _tc_stage(x, [q0, q1], W2)
        q2 = _sc_gather(idx2, W2)
        outs.append((q0 + q1) + q2)
        for j, m in enumerate((m0, m1, m2)):
            msums[j] = msums[j] + m
    out = jnp.concatenate(outs, axis=0)
    a = jnp.stack(msums) * (1.0 / (_N * _D))
    losses = 1.0 * a + 0.25 * a
    return (out, losses)
